# baseline (speedup 1.0000x reference)
import jax
import jax.numpy as jnp
from jax import lax
from jax.experimental import pallas as pl
from jax.experimental.pallas import tpu as pltpu
from jax.experimental.pallas import tpu_sc as plsc

NY, NX, C = 496, 432, 64
P = 100000
BS = 4

NBIN = 2 * NY
NBIN_T = NBIN + 1
NPAD = 1024
SLABS_PER_TILE = NBIN // 16
TPP = P // 16
LPT = 391
NSLOT = 16 * LPT
WQ = 1568
WSTARTS = (0, 1568, 3136, 4688)
NROW = (NSLOT + 127) // 128
HIST_W = 16 * NPAD
BINCAP = 16 * NROW * 128 + 8 * NBIN_T + 256
KCH = 64


def _iota16():
    return lax.broadcasted_iota(jnp.int32, (16,), 0)


def _sc_scatter(coords, feat, out, cblk, s_str, valb, posb, hist, grand_v,
                start_v, base_v, chunk_v, pid_v, pr_v, x_v, rows_v, w0, w1,
                totc, totg, binned, semb, sem0, sem1):
    core = lax.axis_index("c")
    tile = lax.axis_index("s")
    iota = _iota16()
    tbase = tile * TPP
    base8 = pl.multiple_of((tbase // 8) * 8, 8)
    off = tbase - base8

    def sfill_body(i, _):
        s_str[pl.ds(i * 16, 16)] = jnp.full((16,), NBIN, jnp.int32)
        return 0

    lax.fori_loop(0, (NSLOT + 16) // 16, sfill_body, 0)

    for wstart in WSTARTS:
        pltpu.sync_copy(coords.at[pl.ds((base8 + wstart) * 4, 4 * WQ)], cblk)

        def a_body(j, _):
            rows4 = (j * 16 + iota) * 4
            b16 = plsc.load_gather(cblk, [rows4])
            y16 = plsc.load_gather(cblk, [rows4 + 2])
            x16 = plsc.load_gather(cblk, [rows4 + 3])
            q = wstart + j * 16 + iota - off
            valid = (q >= 0) & (q < TPP) & ((b16 >> 1) == core)
            inrange = (q >= 0) & (q < TPP)
            qc = jnp.minimum(jnp.maximum(q, 0), TPP - 1)
            lane = qc // LPT
            v = qc - lane * LPT
            pid = tbase + qc
            s16 = jnp.where(valid, (b16 & 1) * NY + y16, NBIN)
            val16 = (pid << 9) | x16
            pos_s = jnp.where(inrange, v * 16 + lane, NSLOT + iota)
            plsc.store_scatter(s_str, [pos_s], s16)
            row = jnp.where(inrange, v >> 3, NROW)
            col = jnp.where(inrange, (v & 7) * 16 + lane, iota)
            plsc.store_scatter(valb, [row, col], val16)
            return 0

        lax.fori_loop(0, WQ // 16, a_body, 0)

    def z_body(i, _):
        hist[pl.ds(i * 16, 16)] = jnp.zeros((16,), jnp.int32)
        return 0

    lax.fori_loop(0, HIST_W // 16, z_body, 0)
    ones = jnp.ones((16,), jnp.int32)

    def a2_body(v, _):
        s16 = s_str[pl.ds(v * 16, 16)]
        plsc.addupdate_scatter(hist, [s16 * 16 + iota], ones)
        return 0

    lax.fori_loop(0, LPT, a2_body, 0)

    def tot_body(g, _):
        acc = jnp.zeros((16,), jnp.int32)
        for lane in range(16):
            acc = acc + plsc.load_gather(hist, [(g * 16 + iota) * 16 + lane])
        grand_v[pl.ds(g * 16, 16)] = acc
        return 0

    lax.fori_loop(0, NPAD // 16, tot_body, 0)

    pltpu.sync_copy(grand_v, totg.at[tile])
    plsc.subcore_barrier()

    for blk in range(NPAD // 128):
        pltpu.sync_copy(totg.at[:, pl.ds(blk * 128, 128)], totc)

        def grand_body(i, _):
            acc = jnp.zeros((16,), jnp.int32)
            part = jnp.zeros((16,), jnp.int32)
            for t in range(16):
                rowv = totc[t, pl.ds(i * 16, 16)]
                before = (jnp.int32(t) < tile).astype(jnp.int32)
                part = part + rowv * before
                acc = acc + rowv
            grand_v[pl.ds(blk * 128 + i * 16, 16)] = acc
            base_v[pl.ds(blk * 128 + i * 16, 16)] = part
            return 0

        lax.fori_loop(0, 8, grand_body, 0)

    def scan_body(g, run):
        sz = (grand_v[pl.ds(g * 16, 16)] + 7) & ~7
        incl = plsc.cumsum(sz)
        start_v[pl.ds(g * 16, 16)] = incl - sz + run
        return run + jnp.sum(sz, axis=0)

    lax.fori_loop(0, NPAD // 16, scan_body, jnp.int32(0))

    def fold_body(g, _):
        base_v[pl.ds(g * 16, 16)] = (base_v[pl.ds(g * 16, 16)]
                                     + start_v[pl.ds(g * 16, 16)])
        return 0

    lax.fori_loop(0, NPAD // 16, fold_body, 0)

    def next_body(s, _):
        rowv = hist[pl.ds(s * 16, 16)]
        pref = plsc.cumsum(rowv) - rowv
        hist[pl.ds(s * 16, 16)] = pref + base_v[pl.ds(s, 16)][0]
        return 0

    lax.fori_loop(0, NBIN_T, next_body, 0)

    def b_body(v, _):
        s16 = s_str[pl.ds(v * 16, 16)]
        addr = s16 * 16 + iota
        pos = plsc.load_gather(hist, [addr])
        plsc.store_scatter(hist, [addr], pos + 1)
        plsc.store_scatter(posb, [jnp.full((16,), v >> 3, jnp.int32),
                                  (v & 7) * 16 + iota], pos)
        return 0

    lax.fori_loop(0, LPT, b_body, 0)
    plsc.store_scatter(posb, [jnp.full((16,), NROW - 1, jnp.int32),
                              112 + iota],
                       jnp.full((16,), BINCAP - 16, jnp.int32) + iota)
    copies = [pltpu.async_copy(valb.at[r], binned.at[posb.at[r]], semb)
              for r in range(NROW)]
    for cp in copies:
        cp.wait()
    plsc.subcore_barrier()

    def wz(win):
        def wz_body(i, _):
            win[i // (NX // 16), pl.ds((i % (NX // 16)) * 16,
                                       16)] = jnp.zeros((16,), jnp.float32)
            return 0

        lax.fori_loop(0, C * (NX // 16), wz_body, 0)

    wz(w0)
    wz(w1)
    zrow = jnp.zeros((16,), jnp.float32)

    def compose(slab, win, write_feats):
        start = pl.multiple_of(start_v[pl.ds(slab, 16)][0], 8)
        n = grand_v[pl.ds(slab, 16)][0]
        nch = (n + KCH - 1) // KCH

        def ch_body(jj, _):
            pltpu.sync_copy(binned.at[pl.ds(start + jj * KCH, KCH)], chunk_v)
            for k in range(KCH // 16):
                v16 = chunk_v[pl.ds(k * 16, 16)]
                pid_v[pl.ds(k * 16, 16)] = jnp.minimum(
                    jnp.maximum(v16 >> 10, 0), P // 2 - 1)
                pr_v[pl.ds(k * 16, 16)] = (v16 >> 9) & 1
                x_v[pl.ds(k * 16, 16)] = v16 & 511
            if write_feats:
                pltpu.async_copy(feat.at[pid_v], rows_v, semb).wait()
            m = jnp.minimum(n - jj * KCH, KCH)

            def p_body(i, _):
                xv = jnp.full((16,), x_v[pl.ds(i, 16)][0], jnp.int32)
                if write_feats:
                    hi = jnp.full((16,), pr_v[pl.ds(i, 16)][0], jnp.int32) > 0
                for rr in range(4):
                    if write_feats:
                        rv = jnp.where(hi,
                                       rows_v[i, pl.ds(64 + rr * 16, 16)],
                                       rows_v[i, pl.ds(rr * 16, 16)])
                    else:
                        rv = zrow
                    plsc.store_scatter(win, [rr * 16 + iota, xv], rv)
                return 0

            lax.fori_loop(0, m, p_body, 0)
            return 0

        lax.fori_loop(0, nch, ch_body, 0)

    def outslice(slab):
        b_loc = slab // NY
        y = slab % NY
        return out.at[2 * core + b_loc, :, y, :]

    def pair_body(k, _):
        s0 = tile * SLABS_PER_TILE + 2 * k
        s1 = s0 + 1
        compose(s0, w0, True)
        h0 = pltpu.async_copy(w0, outslice(s0), sem0)
        compose(s1, w1, True)
        h1 = pltpu.async_copy(w1, outslice(s1), sem1)
        h0.wait()
        compose(s0, w0, False)
        h1.wait()
        compose(s1, w1, False)
        return 0

    lax.fori_loop(0, SLABS_PER_TILE // 2, pair_body, 0)


def kernel(pillar_features, coords, batch_size):
    f = pl.kernel(
        _sc_scatter,
        out_type=jax.ShapeDtypeStruct((BS, C, NY, NX), jnp.float32),
        mesh=plsc.VectorSubcoreMesh(core_axis_name="c", subcore_axis_name="s"),
        compiler_params=pltpu.CompilerParams(needs_layout_passes=False),
        scratch_types=[
            pltpu.VMEM((4 * WQ,), jnp.int32),
            pltpu.VMEM((NSLOT + 16,), jnp.int32),
            pltpu.VMEM((NROW + 1, 128), jnp.int32),
            pltpu.VMEM((NROW, 128), jnp.int32),
            pltpu.VMEM((HIST_W,), jnp.int32),
            pltpu.VMEM((NPAD,), jnp.int32),
            pltpu.VMEM((NPAD,), jnp.int32),
            pltpu.VMEM((NPAD,), jnp.int32),
            pltpu.VMEM((KCH,), jnp.int32),
            pltpu.VMEM((KCH,), jnp.int32),
            pltpu.VMEM((KCH + 16,), jnp.int32),
            pltpu.VMEM((KCH + 16,), jnp.int32),
            pltpu.VMEM((KCH, 2 * C), jnp.float32),
            pltpu.VMEM((C, NX), jnp.float32),
            pltpu.VMEM((C, NX), jnp.float32),
            pltpu.VMEM((16, 128), jnp.int32),
            pltpu.VMEM_SHARED((16, NPAD), jnp.int32),
            pltpu.VMEM_SHARED((BINCAP,), jnp.int32),
            pltpu.SemaphoreType.DMA,
            pltpu.SemaphoreType.DMA,
            pltpu.SemaphoreType.DMA,
        ],
    )
    return f(coords.astype(jnp.int32).reshape(4 * P),
             pillar_features.reshape(P // 2, 2 * C))


# speedup vs baseline: 1.1178x; 1.1178x over previous
import jax
import jax.numpy as jnp
from jax import lax
from jax.experimental import pallas as pl
from jax.experimental.pallas import tpu as pltpu
from jax.experimental.pallas import tpu_sc as plsc

NY, NX, C = 496, 432, 64
P = 100000
BS = 4

NBIN = 2 * NY
NBIN_T = NBIN + 1
NPAD = 1024
SLABS_PER_TILE = NBIN // 16
TPP = P // 16
LPT = 391
NSLOT = 16 * LPT
WQ = 1568
WSTARTS = (0, 1568, 3136, 4688)
NROW = (NSLOT + 127) // 128
HIST_W = 16 * NPAD
BINCAP = 16 * NROW * 128 + 8 * NBIN_T + 256
KCH = 64


def _iota16():
    return lax.broadcasted_iota(jnp.int32, (16,), 0)


def _sc_scatter(coords, feat, out, cblk, s_str, valb, posb, hist, grand_v,
                start_v, base_v, chunk_v, pid_v, pr_v, x_v, rows_v, w0, w1,
                totc, totg, binned, semb, sem0, sem1):
    core = lax.axis_index("c")
    tile = lax.axis_index("s")
    iota = _iota16()
    tbase = tile * TPP
    base8 = pl.multiple_of((tbase // 8) * 8, 8)
    off = tbase - base8

    def sfill_body(i, _):
        s_str[pl.ds(i * 16, 16)] = jnp.full((16,), NBIN, jnp.int32)
        return 0

    lax.fori_loop(0, (NSLOT + 16) // 16, sfill_body, 0)

    for wstart in WSTARTS:
        pltpu.sync_copy(coords.at[pl.ds((base8 + wstart) * 4, 4 * WQ)], cblk)

        def a_body(j, _):
            rows4 = (j * 16 + iota) * 4
            b16 = plsc.load_gather(cblk, [rows4])
            y16 = plsc.load_gather(cblk, [rows4 + 2])
            x16 = plsc.load_gather(cblk, [rows4 + 3])
            q = wstart + j * 16 + iota - off
            valid = (q >= 0) & (q < TPP) & ((b16 >> 1) == core)
            inrange = (q >= 0) & (q < TPP)
            qc = jnp.minimum(jnp.maximum(q, 0), TPP - 1)
            lane = qc // LPT
            v = qc - lane * LPT
            pid = tbase + qc
            s16 = jnp.where(valid, (b16 & 1) * NY + y16, NBIN)
            val16 = (pid << 9) | x16
            pos_s = jnp.where(inrange, v * 16 + lane, NSLOT + iota)
            plsc.store_scatter(s_str, [pos_s], s16)
            row = jnp.where(inrange, v >> 3, NROW)
            col = jnp.where(inrange, (v & 7) * 16 + lane, iota)
            plsc.store_scatter(valb, [row, col], val16)
            return 0

        lax.fori_loop(0, WQ // 16, a_body, 0)

    def z_body(i, _):
        hist[pl.ds(i * 16, 16)] = jnp.zeros((16,), jnp.int32)
        return 0

    lax.fori_loop(0, HIST_W // 16, z_body, 0)
    ones = jnp.ones((16,), jnp.int32)

    def a2_body(v, _):
        s16 = s_str[pl.ds(v * 16, 16)]
        plsc.addupdate_scatter(hist, [s16 * 16 + iota], ones)
        return 0

    lax.fori_loop(0, LPT, a2_body, 0)

    def tot_body(g, _):
        acc = jnp.zeros((16,), jnp.int32)
        for lane in range(16):
            acc = acc + plsc.load_gather(hist, [(g * 16 + iota) * 16 + lane])
        grand_v[pl.ds(g * 16, 16)] = acc
        return 0

    lax.fori_loop(0, NPAD // 16, tot_body, 0)

    pltpu.sync_copy(grand_v, totg.at[tile])
    plsc.subcore_barrier()

    for blk in range(NPAD // 128):
        pltpu.sync_copy(totg.at[:, pl.ds(blk * 128, 128)], totc)

        def grand_body(i, _):
            acc = jnp.zeros((16,), jnp.int32)
            part = jnp.zeros((16,), jnp.int32)
            for t in range(16):
                rowv = totc[t, pl.ds(i * 16, 16)]
                before = (jnp.int32(t) < tile).astype(jnp.int32)
                part = part + rowv * before
                acc = acc + rowv
            grand_v[pl.ds(blk * 128 + i * 16, 16)] = acc
            base_v[pl.ds(blk * 128 + i * 16, 16)] = part
            return 0

        lax.fori_loop(0, 8, grand_body, 0)

    def scan_body(g, run):
        sz = (grand_v[pl.ds(g * 16, 16)] + 7) & ~7
        incl = plsc.cumsum(sz)
        start_v[pl.ds(g * 16, 16)] = incl - sz + run
        return run + jnp.sum(sz, axis=0)

    lax.fori_loop(0, NPAD // 16, scan_body, jnp.int32(0))

    def fold_body(g, _):
        base_v[pl.ds(g * 16, 16)] = (base_v[pl.ds(g * 16, 16)]
                                     + start_v[pl.ds(g * 16, 16)])
        return 0

    lax.fori_loop(0, NPAD // 16, fold_body, 0)

    def next_body(s, _):
        rowv = hist[pl.ds(s * 16, 16)]
        pref = plsc.cumsum(rowv) - rowv
        hist[pl.ds(s * 16, 16)] = pref + base_v[pl.ds(s, 16)][0]
        return 0

    lax.fori_loop(0, NBIN_T, next_body, 0)

    def b_body(v, _):
        s16 = s_str[pl.ds(v * 16, 16)]
        addr = s16 * 16 + iota
        pos = plsc.load_gather(hist, [addr])
        plsc.store_scatter(hist, [addr], pos + 1)
        plsc.store_scatter(posb, [jnp.full((16,), v >> 3, jnp.int32),
                                  (v & 7) * 16 + iota], pos)
        return 0

    lax.fori_loop(0, LPT, b_body, 0)
    plsc.store_scatter(posb, [jnp.full((16,), NROW - 1, jnp.int32),
                              112 + iota],
                       jnp.full((16,), BINCAP - 16, jnp.int32) + iota)
    copies = [pltpu.async_copy(valb.at[r], binned.at[posb.at[r]], semb)
              for r in range(NROW)]
    for cp in copies:
        cp.wait()
    plsc.subcore_barrier()

    def wz(win):
        def wz_body(i, _):
            win[i // (NX // 16), pl.ds((i % (NX // 16)) * 16,
                                       16)] = jnp.zeros((16,), jnp.float32)
            return 0

        lax.fori_loop(0, C * (NX // 16), wz_body, 0)

    wz(w0)
    wz(w1)
    zrow = jnp.zeros((16,), jnp.float32)

    def compose(slab, win, write_feats):
        start = pl.multiple_of(start_v[pl.ds(slab, 16)][0], 8)
        n = grand_v[pl.ds(slab, 16)][0]
        nch = (n + KCH - 1) // KCH

        def ch_body(jj, _):
            pltpu.sync_copy(binned.at[pl.ds(start + jj * KCH, KCH)], chunk_v)
            for k in range(KCH // 16):
                v16 = chunk_v[pl.ds(k * 16, 16)]
                pid_v[pl.ds(k * 16, 16)] = jnp.minimum(
                    jnp.maximum(v16 >> 10, 0), P // 2 - 1)
                pr_v[pl.ds(k * 16, 16)] = (v16 >> 9) & 1
                x_v[pl.ds(k * 16, 16)] = v16 & 511
            if write_feats:
                pltpu.async_copy(feat.at[pid_v], rows_v, semb).wait()
            m = jnp.minimum(n - jj * KCH, KCH)

            for g in range(KCH // 16):
                gb = g * 16

                @pl.when(m > gb)
                def _():
                    x16 = x_v[pl.ds(gb, 16)]
                    lanevalid = (gb + iota) < m
                    if write_feats:
                        key = jnp.where(lanevalid, x16 * 16 + iota,
                                        8192 + iota)
                        ks, _ = plsc.sort_key_val(key, key)
                        xs = ks >> 4
                        lane = ks & 15
                        xnext = lax.gather(
                            xs, jnp.minimum(iota + 1, 15)[:, None],
                            lax.GatherDimensionNumbers(
                                offset_dims=(), collapsed_slice_dims=(0,),
                                start_index_map=(0,)),
                            slice_sizes=(1,),
                            mode=lax.GatherScatterMode.PROMISE_IN_BOUNDS)
                        keep = ((xs != xnext) | (iota == 15)) & (ks < 8192)
                        rowsel = gb + lane
                        colbase = plsc.load_gather(pr_v, [rowsel]) * 64
                        for c in range(C):
                            vals = plsc.load_gather(rows_v,
                                                    [rowsel, colbase + c])
                            plsc.store_scatter(
                                win, [jnp.full((16,), c, jnp.int32), xs],
                                vals, mask=keep)
                    else:
                        for c in range(C):
                            plsc.store_scatter(
                                win, [jnp.full((16,), c, jnp.int32), x16],
                                zrow, mask=lanevalid)
            return 0

        lax.fori_loop(0, nch, ch_body, 0)

    def outslice(slab):
        b_loc = slab // NY
        y = slab % NY
        return out.at[2 * core + b_loc, :, y, :]

    def pair_body(k, _):
        s0 = tile * SLABS_PER_TILE + 2 * k
        s1 = s0 + 1
        compose(s0, w0, True)
        h0 = pltpu.async_copy(w0, outslice(s0), sem0)
        compose(s1, w1, True)
        h1 = pltpu.async_copy(w1, outslice(s1), sem1)
        h0.wait()
        compose(s0, w0, False)
        h1.wait()
        compose(s1, w1, False)
        return 0

    lax.fori_loop(0, SLABS_PER_TILE // 2, pair_body, 0)


def kernel(pillar_features, coords, batch_size):
    f = pl.kernel(
        _sc_scatter,
        out_type=jax.ShapeDtypeStruct((BS, C, NY, NX), jnp.float32),
        mesh=plsc.VectorSubcoreMesh(core_axis_name="c", subcore_axis_name="s"),
        compiler_params=pltpu.CompilerParams(needs_layout_passes=False),
        scratch_types=[
            pltpu.VMEM((4 * WQ,), jnp.int32),
            pltpu.VMEM((NSLOT + 16,), jnp.int32),
            pltpu.VMEM((NROW + 1, 128), jnp.int32),
            pltpu.VMEM((NROW, 128), jnp.int32),
            pltpu.VMEM((HIST_W,), jnp.int32),
            pltpu.VMEM((NPAD,), jnp.int32),
            pltpu.VMEM((NPAD,), jnp.int32),
            pltpu.VMEM((NPAD,), jnp.int32),
            pltpu.VMEM((KCH,), jnp.int32),
            pltpu.VMEM((KCH,), jnp.int32),
            pltpu.VMEM((KCH,), jnp.int32),
            pltpu.VMEM((KCH,), jnp.int32),
            pltpu.VMEM((KCH, 2 * C), jnp.float32),
            pltpu.VMEM((C, NX), jnp.float32),
            pltpu.VMEM((C, NX), jnp.float32),
            pltpu.VMEM((16, 128), jnp.int32),
            pltpu.VMEM_SHARED((16, NPAD), jnp.int32),
            pltpu.VMEM_SHARED((BINCAP,), jnp.int32),
            pltpu.SemaphoreType.DMA,
            pltpu.SemaphoreType.DMA,
            pltpu.SemaphoreType.DMA,
        ],
    )
    return f(coords.astype(jnp.int32).reshape(4 * P),
             pillar_features.reshape(P // 2, 2 * C))


# speedup vs baseline: 1.1938x; 1.0680x over previous
import jax
import jax.numpy as jnp
from jax import lax
from jax.experimental import pallas as pl
from jax.experimental.pallas import tpu as pltpu
from jax.experimental.pallas import tpu_sc as plsc

NY, NX, C = 496, 432, 64
P = 100000
BS = 4

SLAB_H = 2
SLABS_PER_B = NY // SLAB_H
NBIN = 2 * SLABS_PER_B
NBIN_T = NBIN + 1
NPAD = 512
SLABS_PER_TILE = NBIN // 16
TPP = P // 16
LPT = 391
NSLOT = 16 * LPT
WQ = 1568
WSTARTS = (0, 1568, 3136, 4688)
NROW = (NSLOT + 127) // 128
HIST_W = 16 * NPAD
BINCAP = 16 * NROW * 128 + 8 * NBIN_T + 256
KCH = 80


def _iota16():
    return lax.broadcasted_iota(jnp.int32, (16,), 0)


def _sc_scatter(coords, feat, out, cblk, s_str, valb, posb, hist, grand_v,
                start_v, base_v, chunk_v, pid_v, pr_v, cell_v, rows_v, win,
                totc, totg, binned, semb, sem0, sem1):
    core = lax.axis_index("c")
    tile = lax.axis_index("s")
    iota = _iota16()
    tbase = tile * TPP
    base8 = pl.multiple_of((tbase // 8) * 8, 8)
    off = tbase - base8

    def sfill_body(i, _):
        s_str[pl.ds(i * 16, 16)] = jnp.full((16,), NBIN, jnp.int32)
        return 0

    lax.fori_loop(0, (NSLOT + 16) // 16, sfill_body, 0)

    for wstart in WSTARTS:
        pltpu.sync_copy(coords.at[pl.ds((base8 + wstart) * 4, 4 * WQ)], cblk)

        def a_body(j, _):
            rows4 = (j * 16 + iota) * 4
            b16 = plsc.load_gather(cblk, [rows4])
            y16 = plsc.load_gather(cblk, [rows4 + 2])
            x16 = plsc.load_gather(cblk, [rows4 + 3])
            q = wstart + j * 16 + iota - off
            valid = (q >= 0) & (q < TPP) & ((b16 >> 1) == core)
            inrange = (q >= 0) & (q < TPP)
            qc = jnp.minimum(jnp.maximum(q, 0), TPP - 1)
            lane = qc // LPT
            v = qc - lane * LPT
            pid = tbase + qc
            s16 = jnp.where(valid, (b16 & 1) * SLABS_PER_B + (y16 >> 1),
                            NBIN)
            val16 = (pid << 10) | ((y16 & 1) << 9) | x16
            pos_s = jnp.where(inrange, v * 16 + lane, NSLOT + iota)
            plsc.store_scatter(s_str, [pos_s], s16)
            row = jnp.where(inrange, v >> 3, NROW)
            col = jnp.where(inrange, (v & 7) * 16 + lane, iota)
            plsc.store_scatter(valb, [row, col], val16)
            return 0

        lax.fori_loop(0, WQ // 16, a_body, 0)

    def z_body(i, _):
        hist[pl.ds(i * 16, 16)] = jnp.zeros((16,), jnp.int32)
        return 0

    lax.fori_loop(0, HIST_W // 16, z_body, 0)
    ones = jnp.ones((16,), jnp.int32)

    def a2_body(v, _):
        s16 = s_str[pl.ds(v * 16, 16)]
        plsc.addupdate_scatter(hist, [s16 * 16 + iota], ones)
        return 0

    lax.fori_loop(0, LPT, a2_body, 0)

    def tot_body(g, _):
        acc = jnp.zeros((16,), jnp.int32)
        for lane in range(16):
            acc = acc + plsc.load_gather(hist, [(g * 16 + iota) * 16 + lane])
        grand_v[pl.ds(g * 16, 16)] = acc
        return 0

    lax.fori_loop(0, NPAD // 16, tot_body, 0)

    pltpu.sync_copy(grand_v, totg.at[tile])
    plsc.subcore_barrier()

    for blk in range(NPAD // 128):
        pltpu.sync_copy(totg.at[:, pl.ds(blk * 128, 128)], totc)

        def grand_body(i, _):
            acc = jnp.zeros((16,), jnp.int32)
            part = jnp.zeros((16,), jnp.int32)
            for t in range(16):
                rowv = totc[t, pl.ds(i * 16, 16)]
                before = (jnp.int32(t) < tile).astype(jnp.int32)
                part = part + rowv * before
                acc = acc + rowv
            grand_v[pl.ds(blk * 128 + i * 16, 16)] = acc
            base_v[pl.ds(blk * 128 + i * 16, 16)] = part
            return 0

        lax.fori_loop(0, 8, grand_body, 0)

    def scan_body(g, run):
        sz = (grand_v[pl.ds(g * 16, 16)] + 7) & ~7
        incl = plsc.cumsum(sz)
        start_v[pl.ds(g * 16, 16)] = incl - sz + run
        return run + jnp.sum(sz, axis=0)

    lax.fori_loop(0, NPAD // 16, scan_body, jnp.int32(0))

    def fold_body(g, _):
        base_v[pl.ds(g * 16, 16)] = (base_v[pl.ds(g * 16, 16)]
                                     + start_v[pl.ds(g * 16, 16)])
        return 0

    lax.fori_loop(0, NPAD // 16, fold_body, 0)

    def next_body(s, _):
        rowv = hist[pl.ds(s * 16, 16)]
        pref = plsc.cumsum(rowv) - rowv
        hist[pl.ds(s * 16, 16)] = pref + base_v[pl.ds(s, 16)][0]
        return 0

    lax.fori_loop(0, NBIN_T, next_body, 0)

    def b_body(v, _):
        s16 = s_str[pl.ds(v * 16, 16)]
        addr = s16 * 16 + iota
        pos = plsc.load_gather(hist, [addr])
        plsc.store_scatter(hist, [addr], pos + 1)
        plsc.store_scatter(posb, [jnp.full((16,), v >> 3, jnp.int32),
                                  (v & 7) * 16 + iota], pos)
        return 0

    lax.fori_loop(0, LPT, b_body, 0)
    plsc.store_scatter(posb, [jnp.full((16,), NROW - 1, jnp.int32),
                              112 + iota],
                       jnp.full((16,), BINCAP - 16, jnp.int32) + iota)
    copies = [pltpu.async_copy(valb.at[r], binned.at[posb.at[r]], semb)
              for r in range(NROW)]
    for cp in copies:
        cp.wait()
    plsc.subcore_barrier()

    def wz_body(i, _):
        nxg = NX // 16
        win[i // (C * nxg), (i // nxg) % C,
            pl.ds((i % nxg) * 16, 16)] = jnp.zeros((16,), jnp.float32)
        return 0

    lax.fori_loop(0, SLAB_H * C * (NX // 16), wz_body, 0)
    zrow = jnp.zeros((16,), jnp.float32)

    def compose(slab, write_feats):
        start = pl.multiple_of(start_v[pl.ds(slab, 16)][0], 8)
        n = grand_v[pl.ds(slab, 16)][0]
        nch = (n + KCH - 1) // KCH

        def ch_body(jj, _):
            pltpu.sync_copy(binned.at[pl.ds(start + jj * KCH, KCH)], chunk_v)
            for k in range(KCH // 16):
                v16 = chunk_v[pl.ds(k * 16, 16)]
                pid = jnp.minimum(jnp.maximum(v16 >> 10, 0), P - 1)
                pid_v[pl.ds(k * 16, 16)] = pid >> 1
                pr_v[pl.ds(k * 16, 16)] = pid & 1
                cell_v[pl.ds(k * 16, 16)] = (((v16 >> 9) & 1) * NX
                                             + (v16 & 511))
            if write_feats:
                pltpu.async_copy(feat.at[pid_v], rows_v, semb).wait()
            m = jnp.minimum(n - jj * KCH, KCH)

            for g in range(KCH // 16):
                gb = g * 16

                @pl.when(m > gb)
                def _():
                    c16 = cell_v[pl.ds(gb, 16)]
                    lanevalid = (gb + iota) < m
                    if write_feats:
                        key = jnp.where(lanevalid, c16 * 16 + iota,
                                        16384 + iota)
                        ks, _ = plsc.sort_key_val(key, key)
                        cs = ks >> 4
                        lane = ks & 15
                        cnext = lax.gather(
                            cs, jnp.minimum(iota + 1, 15)[:, None],
                            lax.GatherDimensionNumbers(
                                offset_dims=(), collapsed_slice_dims=(0,),
                                start_index_map=(0,)),
                            slice_sizes=(1,),
                            mode=lax.GatherScatterMode.PROMISE_IN_BOUNDS)
                        keep = ((cs != cnext) | (iota == 15)) & (ks < 16384)
                        yl_s = (cs >= NX).astype(jnp.int32)
                        x_s = cs - yl_s * NX
                        rowsel = gb + lane
                        colbase = plsc.load_gather(pr_v, [rowsel]) * 64
                        for c in range(C):
                            vals = plsc.load_gather(rows_v,
                                                    [rowsel, colbase + c])
                            plsc.store_scatter(
                                win, [yl_s, jnp.full((16,), c, jnp.int32),
                                      x_s], vals, mask=keep)
                    else:
                        yl0 = (c16 >= NX).astype(jnp.int32)
                        x0 = c16 - yl0 * NX
                        for c in range(C):
                            plsc.store_scatter(
                                win, [yl0, jnp.full((16,), c, jnp.int32),
                                      x0], zrow, mask=lanevalid)
            return 0

        lax.fori_loop(0, nch, ch_body, 0)

    def slab_body(sl, _):
        slab = tile * SLABS_PER_TILE + sl
        b_loc = slab // SLABS_PER_B
        r = slab % SLABS_PER_B
        b = 2 * core + b_loc
        compose(slab, True)
        h0 = pltpu.async_copy(win.at[0], out.at[b, :, 2 * r, :], sem0)
        h1 = pltpu.async_copy(win.at[1], out.at[b, :, 2 * r + 1, :], sem1)
        h0.wait()
        h1.wait()
        compose(slab, False)
        return 0

    lax.fori_loop(0, SLABS_PER_TILE, slab_body, 0)


def kernel(pillar_features, coords, batch_size):
    f = pl.kernel(
        _sc_scatter,
        out_type=jax.ShapeDtypeStruct((BS, C, NY, NX), jnp.float32),
        mesh=plsc.VectorSubcoreMesh(core_axis_name="c", subcore_axis_name="s"),
        compiler_params=pltpu.CompilerParams(needs_layout_passes=False),
        scratch_types=[
            pltpu.VMEM((4 * WQ,), jnp.int32),
            pltpu.VMEM((NSLOT + 16,), jnp.int32),
            pltpu.VMEM((NROW + 1, 128), jnp.int32),
            pltpu.VMEM((NROW, 128), jnp.int32),
            pltpu.VMEM((HIST_W,), jnp.int32),
            pltpu.VMEM((NPAD,), jnp.int32),
            pltpu.VMEM((NPAD,), jnp.int32),
            pltpu.VMEM((NPAD,), jnp.int32),
            pltpu.VMEM((KCH,), jnp.int32),
            pltpu.VMEM((KCH,), jnp.int32),
            pltpu.VMEM((KCH,), jnp.int32),
            pltpu.VMEM((KCH,), jnp.int32),
            pltpu.VMEM((KCH, 2 * C), jnp.float32),
            pltpu.VMEM((SLAB_H, C, NX), jnp.float32),
            pltpu.VMEM((16, 128), jnp.int32),
            pltpu.VMEM_SHARED((16, NPAD), jnp.int32),
            pltpu.VMEM_SHARED((BINCAP,), jnp.int32),
            pltpu.SemaphoreType.DMA,
            pltpu.SemaphoreType.DMA,
            pltpu.SemaphoreType.DMA,
        ],
    )
    return f(coords.astype(jnp.int32).reshape(4 * P),
             pillar_features.reshape(P // 2, 2 * C))


# speedup vs baseline: 1.2471x; 1.0447x over previous
import jax
import jax.numpy as jnp
from jax import lax
from jax.experimental import pallas as pl
from jax.experimental.pallas import tpu as pltpu
from jax.experimental.pallas import tpu_sc as plsc

NY, NX, C = 496, 432, 64
P = 100000
BS = 4

SLAB_H = 2
SLABS_PER_B = NY // SLAB_H
NBIN = 2 * SLABS_PER_B
NBIN_T = NBIN + 1
NPAD = 512
SLABS_PER_TILE = NBIN // 16
TPP = P // 16
LPT = 391
NSLOT = 16 * LPT
WQ = 1568
WSTARTS = (0, 1568, 3136, 4688)
NROW = (NSLOT + 127) // 128
HIST_W = 16 * NPAD
BINCAP = 16 * NROW * 128 + 8 * NBIN_T + 256
KCH = 128


def _iota16():
    return lax.broadcasted_iota(jnp.int32, (16,), 0)


def _sc_scatter(coords, feat, out, cblk, s_str, valb, posb, hist, grand_v,
                start_v, base_v, chunk_v, pid_v, pr_v, cell_v, rows_v, win,
                totc, totg, binned, semb, sem0, sem1):
    core = lax.axis_index("c")
    tile = lax.axis_index("s")
    iota = _iota16()
    tbase = tile * TPP
    base8 = pl.multiple_of((tbase // 8) * 8, 8)
    off = tbase - base8

    def sfill_body(i, _):
        s_str[pl.ds(i * 16, 16)] = jnp.full((16,), NBIN, jnp.int32)
        return 0

    lax.fori_loop(0, (NSLOT + 16) // 16, sfill_body, 0)

    for wstart in WSTARTS:
        pltpu.sync_copy(coords.at[pl.ds((base8 + wstart) * 4, 4 * WQ)], cblk)

        def a_body(j, _):
            rows4 = (j * 16 + iota) * 4
            b16 = plsc.load_gather(cblk, [rows4])
            y16 = plsc.load_gather(cblk, [rows4 + 2])
            x16 = plsc.load_gather(cblk, [rows4 + 3])
            q = wstart + j * 16 + iota - off
            valid = (q >= 0) & (q < TPP) & ((b16 >> 1) == core)
            inrange = (q >= 0) & (q < TPP)
            qc = jnp.minimum(jnp.maximum(q, 0), TPP - 1)
            lane = qc // LPT
            v = qc - lane * LPT
            pid = tbase + qc
            s16 = jnp.where(valid, (b16 & 1) * SLABS_PER_B + (y16 >> 1),
                            NBIN)
            val16 = (pid << 10) | ((y16 & 1) << 9) | x16
            pos_s = jnp.where(inrange, v * 16 + lane, NSLOT + iota)
            plsc.store_scatter(s_str, [pos_s], s16)
            row = jnp.where(inrange, v >> 3, NROW)
            col = jnp.where(inrange, (v & 7) * 16 + lane, iota)
            plsc.store_scatter(valb, [row, col], val16)
            return 0

        lax.fori_loop(0, WQ // 16, a_body, 0)

    def z_body(i, _):
        hist[pl.ds(i * 16, 16)] = jnp.zeros((16,), jnp.int32)
        return 0

    lax.fori_loop(0, HIST_W // 16, z_body, 0)
    ones = jnp.ones((16,), jnp.int32)

    def a2_body(v, _):
        s16 = s_str[pl.ds(v * 16, 16)]
        plsc.addupdate_scatter(hist, [s16 * 16 + iota], ones)
        return 0

    lax.fori_loop(0, LPT, a2_body, 0)

    def tot_body(g, _):
        acc = jnp.zeros((16,), jnp.int32)
        for lane in range(16):
            acc = acc + plsc.load_gather(hist, [(g * 16 + iota) * 16 + lane])
        grand_v[pl.ds(g * 16, 16)] = acc
        return 0

    lax.fori_loop(0, NPAD // 16, tot_body, 0)

    pltpu.sync_copy(grand_v, totg.at[tile])
    plsc.subcore_barrier()

    for blk in range(NPAD // 128):
        pltpu.sync_copy(totg.at[:, pl.ds(blk * 128, 128)], totc)

        def grand_body(i, _):
            acc = jnp.zeros((16,), jnp.int32)
            part = jnp.zeros((16,), jnp.int32)
            for t in range(16):
                rowv = totc[t, pl.ds(i * 16, 16)]
                before = (jnp.int32(t) < tile).astype(jnp.int32)
                part = part + rowv * before
                acc = acc + rowv
            grand_v[pl.ds(blk * 128 + i * 16, 16)] = acc
            base_v[pl.ds(blk * 128 + i * 16, 16)] = part
            return 0

        lax.fori_loop(0, 8, grand_body, 0)

    def scan_body(g, run):
        sz = (grand_v[pl.ds(g * 16, 16)] + 7) & ~7
        incl = plsc.cumsum(sz)
        start_v[pl.ds(g * 16, 16)] = incl - sz + run
        return run + jnp.sum(sz, axis=0)

    lax.fori_loop(0, NPAD // 16, scan_body, jnp.int32(0))

    def fold_body(g, _):
        base_v[pl.ds(g * 16, 16)] = (base_v[pl.ds(g * 16, 16)]
                                     + start_v[pl.ds(g * 16, 16)])
        return 0

    lax.fori_loop(0, NPAD // 16, fold_body, 0)

    def next_body(s, _):
        rowv = hist[pl.ds(s * 16, 16)]
        pref = plsc.cumsum(rowv) - rowv
        hist[pl.ds(s * 16, 16)] = pref + base_v[pl.ds(s, 16)][0]
        return 0

    lax.fori_loop(0, NBIN_T, next_body, 0)

    def b_body(v, _):
        s16 = s_str[pl.ds(v * 16, 16)]
        addr = s16 * 16 + iota
        pos = plsc.load_gather(hist, [addr])
        plsc.store_scatter(hist, [addr], pos + 1)
        plsc.store_scatter(posb, [jnp.full((16,), v >> 3, jnp.int32),
                                  (v & 7) * 16 + iota], pos)
        return 0

    lax.fori_loop(0, LPT, b_body, 0)
    plsc.store_scatter(posb, [jnp.full((16,), NROW - 1, jnp.int32),
                              112 + iota],
                       jnp.full((16,), BINCAP - 16, jnp.int32) + iota)
    copies = [pltpu.async_copy(valb.at[r], binned.at[posb.at[r]], semb)
              for r in range(NROW)]
    for cp in copies:
        cp.wait()
    plsc.subcore_barrier()

    def wz_body(i, _):
        nxg = NX // 16
        win[i // (C * nxg), (i // nxg) % C,
            pl.ds((i % nxg) * 16, 16)] = jnp.zeros((16,), jnp.float32)
        return 0

    lax.fori_loop(0, SLAB_H * C * (NX // 16), wz_body, 0)
    zrow = jnp.zeros((16,), jnp.float32)

    def compose(start, n, nch, write_feats):
        def ch_body(jj, _):
            pltpu.sync_copy(binned.at[pl.ds(start + jj * KCH, KCH)], chunk_v)
            for k in range(KCH // 16):
                v16 = chunk_v[pl.ds(k * 16, 16)]
                pid = jnp.minimum(jnp.maximum(v16 >> 10, 0), P - 1)
                pid_v[pl.ds(k * 16, 16)] = pid >> 1
                pr_v[pl.ds(k * 16, 16)] = pid & 1
                cell_v[pl.ds(k * 16, 16)] = (((v16 >> 9) & 1) * NX
                                             + (v16 & 511))
            if write_feats:
                pltpu.async_copy(feat.at[pid_v], rows_v, semb).wait()
            m = jnp.minimum(n - jj * KCH, KCH)

            for g in range(KCH // 16):
                gb = g * 16

                @pl.when(m > gb)
                def _():
                    c16 = cell_v[pl.ds(gb, 16)]
                    lanevalid = (gb + iota) < m
                    if write_feats:
                        key = jnp.where(lanevalid, c16 * 16 + iota,
                                        16384 + iota)
                        ks, _ = plsc.sort_key_val(key, key)
                        cs = ks >> 4
                        lane = ks & 15
                        cnext = lax.gather(
                            cs, jnp.minimum(iota + 1, 15)[:, None],
                            lax.GatherDimensionNumbers(
                                offset_dims=(), collapsed_slice_dims=(0,),
                                start_index_map=(0,)),
                            slice_sizes=(1,),
                            mode=lax.GatherScatterMode.PROMISE_IN_BOUNDS)
                        keep = ((cs != cnext) | (iota == 15)) & (ks < 16384)
                        yl_s = (cs >= NX).astype(jnp.int32)
                        x_s = cs - yl_s * NX
                        rowsel = gb + lane
                        colbase = plsc.load_gather(pr_v, [rowsel]) * 64
                        for c in range(C):
                            vals = plsc.load_gather(rows_v,
                                                    [rowsel, colbase + c])
                            plsc.store_scatter(
                                win, [yl_s, jnp.full((16,), c, jnp.int32),
                                      x_s], vals, mask=keep)
                    else:
                        yl0 = (c16 >= NX).astype(jnp.int32)
                        x0 = c16 - yl0 * NX
                        for c in range(C):
                            plsc.store_scatter(
                                win, [yl0, jnp.full((16,), c, jnp.int32),
                                      x0], zrow, mask=lanevalid)
            return 0

        lax.fori_loop(0, nch, ch_body, 0)

    def zgroups(m):
        for g in range(KCH // 16):
            gb = g * 16

            @pl.when(m > gb)
            def _():
                c16 = cell_v[pl.ds(gb, 16)]
                lanevalid = (gb + iota) < m
                yl0 = (c16 >= NX).astype(jnp.int32)
                x0 = c16 - yl0 * NX
                for c in range(C):
                    plsc.store_scatter(
                        win, [yl0, jnp.full((16,), c, jnp.int32), x0],
                        zrow, mask=lanevalid)

    def slab_body(sl, _):
        slab = tile * SLABS_PER_TILE + sl
        b_loc = slab // SLABS_PER_B
        r = slab % SLABS_PER_B
        b = 2 * core + b_loc
        start = pl.multiple_of(start_v[pl.ds(slab, 16)][0], 8)
        n = grand_v[pl.ds(slab, 16)][0]
        nch = (n + KCH - 1) // KCH
        compose(start, n, nch, True)
        h0 = pltpu.async_copy(win.at[0], out.at[b, :, 2 * r, :], sem0)
        h1 = pltpu.async_copy(win.at[1], out.at[b, :, 2 * r + 1, :], sem1)
        h0.wait()
        h1.wait()

        @pl.when(n > 0)
        def _():
            zgroups(n - (nch - 1) * KCH)

        def rz_body(jj, _):
            pltpu.sync_copy(binned.at[pl.ds(start + jj * KCH, KCH)], chunk_v)
            for k in range(KCH // 16):
                v16 = chunk_v[pl.ds(k * 16, 16)]
                cell_v[pl.ds(k * 16, 16)] = (((v16 >> 9) & 1) * NX
                                             + (v16 & 511))
            zgroups(KCH)
            return 0

        lax.fori_loop(0, nch - 1, rz_body, 0)
        return 0

    lax.fori_loop(0, SLABS_PER_TILE, slab_body, 0)


def kernel(pillar_features, coords, batch_size):
    f = pl.kernel(
        _sc_scatter,
        out_type=jax.ShapeDtypeStruct((BS, C, NY, NX), jnp.float32),
        mesh=plsc.VectorSubcoreMesh(core_axis_name="c", subcore_axis_name="s"),
        compiler_params=pltpu.CompilerParams(needs_layout_passes=False),
        scratch_types=[
            pltpu.VMEM((4 * WQ,), jnp.int32),
            pltpu.VMEM((NSLOT + 16,), jnp.int32),
            pltpu.VMEM((NROW + 1, 128), jnp.int32),
            pltpu.VMEM((NROW, 128), jnp.int32),
            pltpu.VMEM((HIST_W,), jnp.int32),
            pltpu.VMEM((NPAD,), jnp.int32),
            pltpu.VMEM((NPAD,), jnp.int32),
            pltpu.VMEM((NPAD,), jnp.int32),
            pltpu.VMEM((KCH,), jnp.int32),
            pltpu.VMEM((KCH,), jnp.int32),
            pltpu.VMEM((KCH,), jnp.int32),
            pltpu.VMEM((KCH,), jnp.int32),
            pltpu.VMEM((KCH, 2 * C), jnp.float32),
            pltpu.VMEM((SLAB_H, C, NX), jnp.float32),
            pltpu.VMEM((16, 128), jnp.int32),
            pltpu.VMEM_SHARED((16, NPAD), jnp.int32),
            pltpu.VMEM_SHARED((BINCAP,), jnp.int32),
            pltpu.SemaphoreType.DMA,
            pltpu.SemaphoreType.DMA,
            pltpu.SemaphoreType.DMA,
        ],
    )
    return f(coords.astype(jnp.int32).reshape(4 * P),
             pillar_features.reshape(P // 2, 2 * C))
